# baseline (speedup 1.0000x reference)
import functools
import math

import jax
import jax.numpy as jnp
from jax import lax
from jax.experimental import pallas as pl
from jax.experimental.pallas import tpu as pltpu
from jax.experimental.pallas import tpu_sc as plsc

D_MODEL = 128
SCALE = math.sqrt(D_MODEL)

NC = 2
NS = 16
NW = NC * NS

ROWS_PER_CHUNK = 128

NSLOT = 2
P = 2
LAGP = 1


def _make_gather(n_rows):
    chunks_total = n_rows // ROWS_PER_CHUNK
    cpw = chunks_total // NW
    npair = cpw // P
    assert cpw % P == 0 and npair % NSLOT == 0
    mesh = plsc.VectorSubcoreMesh(core_axis_name="c", subcore_axis_name="s")

    @functools.partial(
        pl.kernel,
        out_type=jax.ShapeDtypeStruct((n_rows, D_MODEL), jnp.float32),
        mesh=mesh,
        scratch_types=[
            pltpu.VMEM((cpw, ROWS_PER_CHUNK), jnp.int32),
            pltpu.VMEM((NSLOT, P * ROWS_PER_CHUNK, D_MODEL), jnp.float32),
            [[pltpu.SemaphoreType.DMA] * P] * NSLOT,
            [pltpu.SemaphoreType.DMA] * NSLOT,
        ],
    )
    def gather(table_hbm, idx_hbm, out_hbm, idx_v, rows_v, gsems, psems):
        wid = lax.axis_index("s") * NC + lax.axis_index("c")
        pltpu.sync_copy(idx_hbm.at[pl.ds(wid * cpw, cpw)], idx_v)
        base = wid * cpw

        def start_gathers(p, sl):
            for h in range(P):
                pltpu.async_copy(
                    table_hbm.at[idx_v.at[p * P + h]],
                    rows_v.at[sl, pl.ds(h * ROWS_PER_CHUNK, ROWS_PER_CHUNK)],
                    gsems[sl][h],
                )

        def wait_gathers(sl):
            for h in range(P):
                pltpu.make_async_copy(
                    table_hbm.at[pl.ds(0, ROWS_PER_CHUNK)],
                    rows_v.at[sl, pl.ds(h * ROWS_PER_CHUNK, ROWS_PER_CHUNK)],
                    gsems[sl][h],
                ).wait()

        def start_put(p, sl):
            row0 = (base + p * P) * ROWS_PER_CHUNK
            pltpu.async_copy(
                rows_v.at[sl],
                out_hbm.at[pl.ds(row0, P * ROWS_PER_CHUNK)],
                psems[sl],
            )

        def wait_put(sl):
            pltpu.make_async_copy(
                rows_v.at[sl],
                out_hbm.at[pl.ds(0, P * ROWS_PER_CHUNK)],
                psems[sl],
            ).wait()

        def scale_slot(sl):
            @plsc.parallel_loop(0, P * ROWS_PER_CHUNK, step=1, unroll=4)
            def _(r):
                for c in range(D_MODEL // 16):
                    v = rows_v[sl, r, pl.ds(c * 16, 16)]
                    rows_v[sl, r, pl.ds(c * 16, 16)] = v * SCALE

        def super_body(pp, carry):
            for u in range(NSLOT):
                p = pp * NSLOT + u

                @pl.when(p >= NSLOT)
                def _():
                    wait_put(u)

                start_gathers(p, u)
                u2 = (u - LAGP) % NSLOT

                @pl.when(p >= LAGP)
                def _():
                    wait_gathers(u2)
                    scale_slot(u2)
                    start_put(p - LAGP, u2)

            return carry

        lax.fori_loop(0, npair // NSLOT, super_body, 0)
        for t in range(LAGP):
            p2 = npair - LAGP + t
            sl2 = p2 % NSLOT
            wait_gathers(sl2)
            scale_slot(sl2)
            start_put(p2, sl2)
        for sl in range(NSLOT):
            wait_put(sl)

    return gather


@jax.jit
def kernel(x, table):
    n_rows = x.size
    xf = x.reshape(n_rows // ROWS_PER_CHUNK, ROWS_PER_CHUNK).astype(jnp.int32)
    out = _make_gather(n_rows)(table, xf)
    return out.reshape(x.shape + (D_MODEL,))
